# baseline (speedup 1.0000x reference)
import jax
import jax.numpy as jnp
from jax import lax
from jax.experimental import pallas as pl
from jax.experimental.pallas import tpu as pltpu
from jax.experimental.pallas import tpu_sc as plsc

B = 16384
H = 50
D = 70
DP = 80
NVOC = 100000

NC = 2
NS = 16
NW = NC * NS
BPW = B // NW
CH = 16
NCHUNK = BPW // CH
GS = 80
NGAT = (CH * H) // GS
NLANE = DP // 16
INV_H = 1.0 / H


def _sc_pool_body(x_hbm, tab_hbm, out_hbm, idx_v, rows_v, pooled_v, sem):
    c = lax.axis_index("c")
    s = lax.axis_index("s")
    wid = s * NC + c
    base = wid * BPW

    def chunk_body(ci, carry):
        b0 = base + ci * CH
        pltpu.sync_copy(x_hbm.at[pl.ds(b0 * H, CH * H)], idx_v)
        descs = [
            pltpu.async_copy(
                tab_hbm.at[idx_v.at[pl.ds(g * GS, GS)]],
                rows_v.at[pl.ds(g * GS, GS)],
                sem,
            )
            for g in range(NGAT)
        ]
        for d_ in descs:
            d_.wait()

        def elem_body(i, carry2):
            r0 = i * H

            def red_body(r, accs):
                row = r0 + r
                return tuple(
                    accs[d] + rows_v[row, pl.ds(16 * d, 16)]
                    for d in range(NLANE)
                )

            accs = tuple(jnp.zeros((16,), jnp.float32) for _ in range(NLANE))
            accs = lax.fori_loop(0, H, red_body, accs)
            for d in range(NLANE):
                pooled_v[i, pl.ds(16 * d, 16)] = accs[d] * INV_H
            return carry2

        lax.fori_loop(0, CH, elem_body, 0)
        pltpu.sync_copy(pooled_v, out_hbm.at[pl.ds(b0, CH)])
        return carry

    lax.fori_loop(0, NCHUNK, chunk_body, 0)


_sc_pool = pl.kernel(
    _sc_pool_body,
    out_type=jax.ShapeDtypeStruct((B, DP), jnp.float32),
    mesh=plsc.VectorSubcoreMesh(
        core_axis_name="c", subcore_axis_name="s", num_cores=NC, num_subcores=NS
    ),
    scratch_types=[
        pltpu.VMEM((CH * H,), jnp.int32),
        pltpu.VMEM((CH * H, DP), jnp.float32),
        pltpu.VMEM((CH, DP), jnp.float32),
        pltpu.SemaphoreType.DMA,
    ],
    compiler_params=pltpu.CompilerParams(use_tc_tiling_on_sc=False),
)


TM = 1024
TN = 1024
NB = B // TM
NVT = pl.cdiv(NVOC, TN)


def _mm_body(m_ref, w_ref, b_ref, out_ref):
    ib = pl.program_id(1)
    mblk = m_ref[pl.ds(ib * TM, TM), :]
    acc = lax.dot_general(
        mblk, w_ref[...], (((1,), (1,)), ((), ())),
        preferred_element_type=jnp.float32,
    )
    out_ref[...] = acc + b_ref[...]


def _tc_matmul(m, w, bias2d):
    return pl.pallas_call(
        _mm_body,
        grid=(NVT, NB),
        in_specs=[
            pl.BlockSpec((B, D), lambda v, b: (0, 0)),
            pl.BlockSpec((TN, D), lambda v, b: (v, 0)),
            pl.BlockSpec((1, TN), lambda v, b: (0, v)),
        ],
        out_specs=pl.BlockSpec((TM, TN), lambda v, b: (b, v)),
        out_shape=jax.ShapeDtypeStruct((B, NVOC), jnp.float32),
    )(m, w, bias2d)


def kernel(x, embed_weight, lin_weight, lin_bias):
    tab = jnp.pad(embed_weight, ((0, 0), (0, DP - D)))
    xf = x.reshape(-1).astype(jnp.int32)
    pooled = _sc_pool(xf, tab)
    m = pooled[:, :D]
    return _tc_matmul(m, lin_weight, lin_bias.reshape(1, NVOC))


# speedup vs baseline: 3.0765x; 3.0765x over previous
import jax
import jax.numpy as jnp
from jax import lax
from jax.experimental import pallas as pl
from jax.experimental.pallas import tpu as pltpu
from jax.experimental.pallas import tpu_sc as plsc

B = 16384
H = 50
D = 70
DP = 80
NVOC = 100000

NC = 2
NS = 16
NW = NC * NS
BPW = B // NW
CH = 16
NCHUNK = BPW // CH
GS = 80
NGAT = (CH * H) // GS
NLANE = DP // 16
INV_H = 1.0 / H


def _sc_pool_body(x_hbm, tab_hbm, out_hbm, idx_v, rows_v, pooled_v, sem):
    c = lax.axis_index("c")
    s = lax.axis_index("s")
    wid = s * NC + c
    base = wid * BPW

    def chunk_body(ci, carry):
        b0 = base + ci * CH
        pltpu.sync_copy(x_hbm.at[pl.ds(b0 * H, CH * H)], idx_v)
        descs = [
            pltpu.async_copy(
                tab_hbm.at[idx_v.at[pl.ds(g * GS, GS)]],
                rows_v.at[pl.ds(g * GS, GS)],
                sem,
            )
            for g in range(NGAT)
        ]
        for d_ in descs:
            d_.wait()

        def elem_body(i, carry2):
            r0 = i * H

            def red_body(r, accs):
                row = r0 + r
                return tuple(
                    accs[d] + rows_v[row, pl.ds(16 * d, 16)]
                    for d in range(NLANE)
                )

            accs = tuple(jnp.zeros((16,), jnp.float32) for _ in range(NLANE))
            accs = lax.fori_loop(0, H, red_body, accs)
            for d in range(NLANE):
                pooled_v[i, pl.ds(16 * d, 16)] = accs[d] * INV_H
            return carry2

        lax.fori_loop(0, CH, elem_body, 0)
        pltpu.sync_copy(pooled_v, out_hbm.at[pl.ds(b0, CH)])
        return carry

    lax.fori_loop(0, NCHUNK, chunk_body, 0)


_sc_pool = pl.kernel(
    _sc_pool_body,
    out_type=jax.ShapeDtypeStruct((B, DP), jnp.float32),
    mesh=plsc.VectorSubcoreMesh(
        core_axis_name="c", subcore_axis_name="s", num_cores=NC, num_subcores=NS
    ),
    scratch_types=[
        pltpu.VMEM((CH * H,), jnp.int32),
        pltpu.VMEM((CH * H, DP), jnp.float32),
        pltpu.VMEM((CH, DP), jnp.float32),
        pltpu.SemaphoreType.DMA,
    ],
    compiler_params=pltpu.CompilerParams(use_tc_tiling_on_sc=False),
)


TNR = 2000
TMC = 2048
NVR = NVOC // TNR
NBC = B // TMC


def _mmt_body(w_ref, m_ref, out_ref):
    out_ref[...] = lax.dot_general(
        w_ref[...], m_ref[...], (((1,), (1,)), ((), ())),
        preferred_element_type=jnp.float32,
    )


def _tc_matmul_t(w_aug, m):
    return pl.pallas_call(
        _mmt_body,
        grid=(NVR, NBC),
        in_specs=[
            pl.BlockSpec((TNR, DP), lambda v, b: (v, 0)),
            pl.BlockSpec((TMC, DP), lambda v, b: (b, 0)),
        ],
        out_specs=pl.BlockSpec((TNR, TMC), lambda v, b: (v, b)),
        out_shape=jax.ShapeDtypeStruct((NVOC, B), jnp.float32),
        compiler_params=pltpu.CompilerParams(
            vmem_limit_bytes=50 * 1024 * 1024
        ),
    )(w_aug, m)


def kernel(x, embed_weight, lin_weight, lin_bias):
    tab = jnp.concatenate(
        [
            embed_weight,
            jnp.ones((NVOC, 1), jnp.float32),
            jnp.zeros((NVOC, DP - D - 1), jnp.float32),
        ],
        axis=1,
    )
    xf = x.reshape(-1).astype(jnp.int32)
    pooled = _sc_pool(xf, tab)
    w_aug = jnp.concatenate(
        [
            lin_weight,
            lin_bias[:, None],
            jnp.zeros((NVOC, DP - D - 1), jnp.float32),
        ],
        axis=1,
    )
    out_t = _tc_matmul_t(w_aug, pooled)
    return out_t.T
